# baseline (speedup 1.0000x reference)
import functools

import jax
import jax.numpy as jnp
from jax import lax
from jax.experimental import pallas as pl
from jax.experimental.pallas import tpu as pltpu
from jax.experimental.pallas import tpu_sc as plsc

_L = 16


def _sc_inner_products(z, src_idx, dst_idx):
    nc, ns = 2, 16
    nw = nc * ns
    e_total = src_idx.shape[0]
    d = z.shape[1]
    ew = e_total // nw
    assert ew * nw == e_total and ew % 8 == 0
    chunk = 80
    nslot = 3
    nchunk = ew // chunk
    assert nchunk * chunk == ew and chunk % _L == 0

    @functools.partial(
        pl.kernel,
        out_type=jax.ShapeDtypeStruct((e_total,), jnp.float32),
        mesh=plsc.VectorSubcoreMesh(core_axis_name="c", subcore_axis_name="s"),
        compiler_params=pltpu.CompilerParams(needs_layout_passes=False),
        scratch_types=[
            pltpu.VMEM((ew,), jnp.int32),
            pltpu.VMEM((ew,), jnp.int32),
            pltpu.VMEM((nslot, chunk, d), jnp.float32),
            pltpu.VMEM((nslot, chunk, d), jnp.float32),
            pltpu.VMEM((ew,), jnp.float32),
            [pltpu.SemaphoreType.DMA] * nslot,
            [pltpu.SemaphoreType.DMA] * nslot,
        ],
    )
    def body(z_hbm, sidx_hbm, didx_hbm, out_hbm,
             sidx_v, didx_v, srows_v, drows_v, out_v, sems_s, sems_d):
        wid = lax.axis_index("s") * nc + lax.axis_index("c")
        base = wid * ew
        pltpu.sync_copy(sidx_hbm.at[pl.ds(base, ew)], sidx_v)
        pltpu.sync_copy(didx_hbm.at[pl.ds(base, ew)], didx_v)
        lanes = lax.iota(jnp.int32, _L)

        def descs(c, sl):
            cs = pltpu.make_async_copy(
                z_hbm.at[sidx_v.at[pl.ds(c * chunk, chunk)]],
                srows_v.at[sl], sems_s[sl])
            cd = pltpu.make_async_copy(
                z_hbm.at[didx_v.at[pl.ds(c * chunk, chunk)]],
                drows_v.at[sl], sems_d[sl])
            return cs, cd

        def start(c, sl):
            cs, cd = descs(c, sl)
            cs.start()
            cd.start()

        def compute(c, sl):
            cs, cd = descs(c, sl)
            cs.wait()
            cd.wait()
            sref = srows_v.at[sl]
            dref = drows_v.at[sl]

            def do_group(g, carry2):
                rows = g * _L + lanes
                col0 = jnp.zeros((_L,), jnp.int32)
                s = plsc.load_gather(sref, [rows, col0])
                dd = plsc.load_gather(dref, [rows, col0])
                zero = jnp.zeros((_L,), jnp.float32)
                accs = [-(s * dd), zero, zero, zero]
                four = jnp.full((_L,), 4, jnp.int32)
                cols = [jnp.full((_L,), j, jnp.int32) for j in (1, 2, 3, 4)]
                for f in range(1, d):
                    j = (f - 1) % 4
                    cf = cols[j]
                    s = plsc.load_gather(sref, [rows, cf])
                    dd = plsc.load_gather(dref, [rows, cf])
                    accs[j] = accs[j] + s * dd
                    cols[j] = cf + four
                acc = (accs[0] + accs[1]) + (accs[2] + accs[3])
                out_v[pl.ds(c * chunk + g * _L, _L)] = acc
                return carry2

            lax.fori_loop(0, chunk // _L, do_group, 0)

        for c in range(nslot - 1):
            start(c, c)

        def step(i, carry):
            for b in range(nslot):
                c = nslot * i + b

                next_slot = (b + nslot - 1) % nslot

                @pl.when(c + nslot - 1 < nchunk)
                def _():
                    start(c + nslot - 1, next_slot)

                compute(c, b)
            return carry

        nfull = nchunk // nslot
        lax.fori_loop(0, nfull, step, 0)
        for c in range(nfull * nslot, nchunk):
            compute(c, c % nslot)

        pltpu.sync_copy(out_v, out_hbm.at[pl.ds(base, ew)])

    return body(z, src_idx, dst_idx)


def _tc_decode_body(r_ref, t_ref, inner_ref, o_ref):
    inner = inner_ref[...]
    arg = jnp.maximum(-inner, 1.0 + 1e-7)
    dist = jnp.log(arg + jnp.sqrt(arg * arg - 1.0))
    o_ref[...] = 1.0 / (jnp.exp((dist - r_ref[0, 0]) / t_ref[0, 0]) + 1.0)


def _tc_decode(inner2d, r, t):
    rows, cols = inner2d.shape
    return pl.pallas_call(
        _tc_decode_body,
        out_shape=jax.ShapeDtypeStruct((rows, cols), jnp.float32),
        in_specs=[
            pl.BlockSpec(memory_space=pltpu.SMEM),
            pl.BlockSpec(memory_space=pltpu.SMEM),
            pl.BlockSpec(memory_space=pltpu.VMEM),
        ],
        out_specs=pl.BlockSpec(memory_space=pltpu.VMEM),
    )(r.reshape(1, 1).astype(jnp.float32), t.reshape(1, 1).astype(jnp.float32),
      inner2d)


def kernel(z, edge_index, r, t):
    ei = edge_index.astype(jnp.int32)
    inner = _sc_inner_products(z, ei[0], ei[1])
    e_total = inner.shape[0]
    probs2d = _tc_decode(inner.reshape(e_total // 128, 128), r, t)
    return probs2d.reshape(e_total)


# speedup vs baseline: 5.5577x; 5.5577x over previous
import functools

import jax
import jax.numpy as jnp
from jax import lax
from jax.experimental import pallas as pl
from jax.experimental.pallas import tpu as pltpu
from jax.experimental.pallas import tpu_sc as plsc

_L = 16


def _sc_fermi_dirac(zp, src_idx, dst_idx, rot, invt):
    nc, ns = 2, 16
    nw = nc * ns
    e_total = src_idx.shape[0]
    dw = zp.shape[1]
    ew = e_total // nw
    assert ew * nw == e_total and ew % 8 == 0
    chunk = 80
    nslot = 4
    nchunk = ew // chunk
    assert nchunk * chunk == ew and chunk % _L == 0

    @functools.partial(
        pl.kernel,
        out_type=jax.ShapeDtypeStruct((e_total,), jnp.float32),
        mesh=plsc.VectorSubcoreMesh(core_axis_name="c", subcore_axis_name="s"),
        compiler_params=pltpu.CompilerParams(
            needs_layout_passes=False, use_tc_tiling_on_sc=False),
        scratch_types=[
            pltpu.VMEM((ew,), jnp.int32),
            pltpu.VMEM((ew,), jnp.int32),
            pltpu.VMEM((nslot, chunk, dw), jnp.int32),
            pltpu.VMEM((nslot, chunk, dw), jnp.int32),
            pltpu.VMEM((ew,), jnp.float32),
            pltpu.VMEM((_L, _L + 1), jnp.float32),
            pltpu.VMEM((_L,), jnp.float32),
            pltpu.VMEM((_L,), jnp.float32),
            [pltpu.SemaphoreType.DMA] * nslot,
            [pltpu.SemaphoreType.DMA] * nslot,
        ],
    )
    def body(z_hbm, sidx_hbm, didx_hbm, rot_hbm, invt_hbm, out_hbm,
             sidx_v, didx_v, srows_v, drows_v, out_v, tbuf_v, rot_v, invt_v,
             sems_s, sems_d):
        wid = lax.axis_index("s") * nc + lax.axis_index("c")
        base = wid * ew
        pltpu.sync_copy(sidx_hbm.at[pl.ds(base, ew)], sidx_v)
        pltpu.sync_copy(didx_hbm.at[pl.ds(base, ew)], didx_v)
        pltpu.sync_copy(rot_hbm, rot_v)
        pltpu.sync_copy(invt_hbm, invt_v)
        lanes = lax.iota(jnp.int32, _L)
        mneg = jnp.where(lanes == 0, -1.0, 1.0).astype(jnp.float32)

        def decode(res):
            arg = jnp.maximum(-res, 1.0 + 1e-7)
            y = arg * arg - 1.0
            h = plsc.bitcast(
                jnp.int32(0x5F3759DF) - (plsc.bitcast(y, jnp.int32) >> 1),
                jnp.float32)
            for _ in range(3):
                h = h * (1.5 - 0.5 * y * h * h)
            u = arg + y * h
            ui = plsc.bitcast(u, jnp.int32)
            ex = (ui >> 23) - 127
            m = plsc.bitcast((ui & 0x007FFFFF) | 0x3F800000, jnp.float32)
            big = m > 1.4142135623730951
            m = jnp.where(big, m * 0.5, m)
            ex = (ex + jnp.where(big, 1, 0)).astype(jnp.float32)
            q = (m - 1.0) / (m + 1.0)
            q2 = q * q
            p = (1.0 / 9.0)
            p = p * q2 + (1.0 / 7.0)
            p = p * q2 + 0.2
            p = p * q2 + (1.0 / 3.0)
            p = p * q2 + 1.0
            dist = ex * 0.6931471805599453 + 2.0 * q * p
            w = jnp.exp(dist * invt_v[...] - rot_v[...])
            return 1.0 / (w + 1.0)

        def descs(c, sl):
            cs = pltpu.make_async_copy(
                z_hbm.at[sidx_v.at[pl.ds(c * chunk, chunk)]],
                srows_v.at[sl], sems_s[sl])
            cd = pltpu.make_async_copy(
                z_hbm.at[didx_v.at[pl.ds(c * chunk, chunk)]],
                drows_v.at[sl], sems_d[sl])
            return cs, cd

        def start(c, sl):
            cs, cd = descs(c, sl)
            cs.start()
            cd.start()

        def compute(c, sl):
            cs, cd = descs(c, sl)
            cs.wait()
            cd.wait()
            sref = srows_v.at[sl]
            dref = drows_v.at[sl]

            def do_group(g, carry2):
                ebase = g * _L
                for l in range(_L):
                    e = ebase + l
                    ps = []
                    for k in range(dw // _L):
                        sv = sref[e, pl.ds(k * _L, _L)]
                        dv = dref[e, pl.ds(k * _L, _L)]
                        pb = (plsc.bitcast(sv, jnp.bfloat16)
                              * plsc.bitcast(dv, jnp.bfloat16))
                        pa, pc = plsc.unpack(
                            pb, format=plsc.PackFormat.INTERLEAVED)
                        if k == 0:
                            pa = pa * mneg
                        ps += [pa, pc]
                    while len(ps) > 1:
                        ps = [ps[i] + ps[i + 1] for i in range(0, len(ps), 2)]
                    tbuf_v[l, pl.ds(0, _L)] = ps[0]
                one = jnp.full((_L,), 1, jnp.int32)
                col = jnp.zeros((_L,), jnp.int32)
                zero = jnp.zeros((_L,), jnp.float32)
                accs = [zero, zero, zero, zero]
                for k in range(_L):
                    accs[k % 4] = accs[k % 4] + plsc.load_gather(
                        tbuf_v, [lanes, col])
                    col = col + one
                res = (accs[0] + accs[1]) + (accs[2] + accs[3])
                out_v[pl.ds(c * chunk + ebase, _L)] = decode(res)
                return carry2

            lax.fori_loop(0, chunk // _L, do_group, 0)

        for c in range(nslot - 1):
            start(c, c)

        def step(i, carry):
            for b in range(nslot):
                c = nslot * i + b

                next_slot = (b + nslot - 1) % nslot

                @pl.when(c + nslot - 1 < nchunk)
                def _():
                    start(c + nslot - 1, next_slot)

                compute(c, b)
            return carry

        nfull = nchunk // nslot
        lax.fori_loop(0, nfull, step, 0)
        for c in range(nfull * nslot, nchunk):
            compute(c, c % nslot)

        pltpu.sync_copy(out_v, out_hbm.at[pl.ds(base, ew)])

    return body(zp, src_idx, dst_idx, rot, invt)


def kernel(z, edge_index, r, t):
    ei = edge_index.astype(jnp.int32)
    n, d = z.shape
    zp = jax.lax.bitcast_convert_type(
        z.astype(jnp.bfloat16).reshape(n, d // 2, 2), jnp.int32)
    rf = r.astype(jnp.float32)
    tf = t.astype(jnp.float32)
    rot = jnp.full((_L,), rf / tf, jnp.float32)
    invt = jnp.full((_L,), 1.0 / tf, jnp.float32)
    return _sc_fermi_dirac(zp, ei[0], ei[1], rot, invt)


# speedup vs baseline: 6.3975x; 1.1511x over previous
import functools

import jax
import jax.numpy as jnp
from jax import lax
from jax.experimental import pallas as pl
from jax.experimental.pallas import tpu as pltpu
from jax.experimental.pallas import tpu_sc as plsc

_L = 16


def _sc_fermi_dirac(zp, src_idx, dst_idx, rot, invt):
    nc, ns = 2, 16
    nw = nc * ns
    e_total = src_idx.shape[0]
    dw = zp.shape[1]
    ew = e_total // nw
    assert ew * nw == e_total and ew % 8 == 0
    chunk = 80
    nslot = 4
    nchunk = ew // chunk
    assert nchunk * chunk == ew and chunk % _L == 0

    @functools.partial(
        pl.kernel,
        out_type=jax.ShapeDtypeStruct((e_total,), jnp.float32),
        mesh=plsc.VectorSubcoreMesh(core_axis_name="c", subcore_axis_name="s"),
        compiler_params=pltpu.CompilerParams(
            needs_layout_passes=False, use_tc_tiling_on_sc=False),
        scratch_types=[
            pltpu.VMEM((ew,), jnp.int32),
            pltpu.VMEM((ew,), jnp.int32),
            pltpu.VMEM((nslot, chunk, dw), jnp.int32),
            pltpu.VMEM((nslot, chunk, dw), jnp.int32),
            pltpu.VMEM((ew,), jnp.float32),
            pltpu.VMEM((_L, _L + 1), jnp.float32),
            pltpu.VMEM((_L,), jnp.float32),
            pltpu.VMEM((_L,), jnp.float32),
            [pltpu.SemaphoreType.DMA] * nslot,
            [pltpu.SemaphoreType.DMA] * nslot,
        ],
    )
    def body(z_hbm, sidx_hbm, didx_hbm, rot_hbm, invt_hbm, out_hbm,
             sidx_v, didx_v, srows_v, drows_v, out_v, tbuf_v, rot_v, invt_v,
             sems_s, sems_d):
        wid = lax.axis_index("s") * nc + lax.axis_index("c")
        base = wid * ew
        pltpu.sync_copy(sidx_hbm.at[pl.ds(base, ew)], sidx_v)
        pltpu.sync_copy(didx_hbm.at[pl.ds(base, ew)], didx_v)
        pltpu.sync_copy(rot_hbm, rot_v)
        pltpu.sync_copy(invt_hbm, invt_v)
        lanes = lax.iota(jnp.int32, _L)
        mneg = jnp.where(lanes == 0, -1.0, 1.0).astype(jnp.float32)

        def decode(res):
            arg = jnp.maximum(-res, 1.0 + 1e-7)
            y = arg * arg - 1.0
            h = plsc.bitcast(
                jnp.int32(0x5F3759DF) - (plsc.bitcast(y, jnp.int32) >> 1),
                jnp.float32)
            for _ in range(3):
                h = h * (1.5 - 0.5 * y * h * h)
            u = arg + y * h
            ui = plsc.bitcast(u, jnp.int32)
            ex = (ui >> 23) - 127
            m = plsc.bitcast((ui & 0x007FFFFF) | 0x3F800000, jnp.float32)
            big = m > 1.4142135623730951
            m = jnp.where(big, m * 0.5, m)
            ex = (ex + jnp.where(big, 1, 0)).astype(jnp.float32)
            q = (m - 1.0) / (m + 1.0)
            q2 = q * q
            p = (1.0 / 9.0)
            p = p * q2 + (1.0 / 7.0)
            p = p * q2 + 0.2
            p = p * q2 + (1.0 / 3.0)
            p = p * q2 + 1.0
            dist = ex * 0.6931471805599453 + 2.0 * q * p
            w = jnp.exp(dist * invt_v[...] - rot_v[...])
            return 1.0 / (w + 1.0)

        def descs(c, sl):
            cs = pltpu.make_async_copy(
                z_hbm.at[sidx_v.at[pl.ds(c * chunk, chunk)]],
                srows_v.at[sl], sems_s[sl])
            cd = pltpu.make_async_copy(
                z_hbm.at[didx_v.at[pl.ds(c * chunk, chunk)]],
                drows_v.at[sl], sems_d[sl])
            return cs, cd

        def start(c, sl):
            cs, cd = descs(c, sl)
            cs.start()
            cd.start()

        def compute(c, sl):
            cs, cd = descs(c, sl)
            cs.wait()
            cd.wait()
            sref = srows_v.at[sl]
            dref = drows_v.at[sl]

            def do_group(g, carry2):
                ebase = g * _L
                for l in range(_L):
                    e = ebase + l
                    ps = []
                    for k in range(dw // _L):
                        sv = sref[e, pl.ds(k * _L, _L)]
                        dv = dref[e, pl.ds(k * _L, _L)]
                        pb = (plsc.bitcast(sv, jnp.bfloat16)
                              * plsc.bitcast(dv, jnp.bfloat16))
                        pa, pc = plsc.unpack(
                            pb, format=plsc.PackFormat.INTERLEAVED)
                        if k == 0:
                            pa = pa * mneg
                        ps += [pa, pc]
                    while len(ps) > 1:
                        ps = [ps[i] + ps[i + 1] for i in range(0, len(ps), 2)]
                    tbuf_v[l, pl.ds(0, _L)] = ps[0]
                one = jnp.full((_L,), 1, jnp.int32)
                col = jnp.zeros((_L,), jnp.int32)
                zero = jnp.zeros((_L,), jnp.float32)
                accs = [zero, zero, zero, zero]
                for k in range(_L):
                    accs[k % 4] = accs[k % 4] + plsc.load_gather(
                        tbuf_v, [lanes, col])
                    col = col + one
                res = (accs[0] + accs[1]) + (accs[2] + accs[3])
                out_v[pl.ds(c * chunk + ebase, _L)] = res
                return carry2

            lax.fori_loop(0, chunk // _L, do_group, 0)

        for c in range(nslot - 1):
            start(c, c)

        def step(i, carry):
            for b in range(nslot):
                c = nslot * i + b

                next_slot = (b + nslot - 1) % nslot

                @pl.when(c + nslot - 1 < nchunk)
                def _():
                    start(c + nslot - 1, next_slot)

                compute(c, b)
            return carry

        nfull = nchunk // nslot
        lax.fori_loop(0, nfull, step, 0)
        for c in range(nfull * nslot, nchunk):
            compute(c, c % nslot)

        dec_unroll = 5
        ngrp = ew // _L

        def dec_step(i, carry):
            for b in range(dec_unroll):
                off = (i * dec_unroll + b) * _L
                out_v[pl.ds(off, _L)] = decode(out_v[pl.ds(off, _L)])
            return carry

        assert ngrp % dec_unroll == 0
        lax.fori_loop(0, ngrp // dec_unroll, dec_step, 0)

        pltpu.sync_copy(out_v, out_hbm.at[pl.ds(base, ew)])

    return body(zp, src_idx, dst_idx, rot, invt)


def kernel(z, edge_index, r, t):
    ei = edge_index.astype(jnp.int32)
    n, d = z.shape
    zp = jax.lax.bitcast_convert_type(
        z.astype(jnp.bfloat16).reshape(n, d // 2, 2), jnp.int32)
    rf = r.astype(jnp.float32)
    tf = t.astype(jnp.float32)
    rot = jnp.full((_L,), rf / tf, jnp.float32)
    invt = jnp.full((_L,), 1.0 / tf, jnp.float32)
    return _sc_fermi_dirac(zp, ei[0], ei[1], rot, invt)


# speedup vs baseline: 6.7812x; 1.0600x over previous
import functools

import jax
import jax.numpy as jnp
from jax import lax
from jax.experimental import pallas as pl
from jax.experimental.pallas import tpu as pltpu
from jax.experimental.pallas import tpu_sc as plsc

_L = 16


def _sc_inner_products(zp, src_idx, dst_idx):
    nc, ns = 2, 16
    nw = nc * ns
    e_total = src_idx.shape[0]
    dw = zp.shape[1]
    ew = e_total // nw
    assert ew * nw == e_total and ew % 8 == 0
    chunk = 80
    nslot = 4
    nchunk = ew // chunk
    assert nchunk * chunk == ew and chunk % _L == 0

    @functools.partial(
        pl.kernel,
        out_type=jax.ShapeDtypeStruct((e_total,), jnp.float32),
        mesh=plsc.VectorSubcoreMesh(core_axis_name="c", subcore_axis_name="s"),
        compiler_params=pltpu.CompilerParams(
            needs_layout_passes=False, use_tc_tiling_on_sc=False),
        scratch_types=[
            pltpu.VMEM((ew,), jnp.int32),
            pltpu.VMEM((ew,), jnp.int32),
            pltpu.VMEM((nslot, chunk, dw), jnp.int32),
            pltpu.VMEM((nslot, chunk, dw), jnp.int32),
            pltpu.VMEM((ew,), jnp.float32),
            pltpu.VMEM((_L, _L + 1), jnp.float32),
            [pltpu.SemaphoreType.DMA] * nslot,
            [pltpu.SemaphoreType.DMA] * nslot,
        ],
    )
    def body(z_hbm, sidx_hbm, didx_hbm, out_hbm,
             sidx_v, didx_v, srows_v, drows_v, out_v, tbuf_v, sems_s, sems_d):
        wid = lax.axis_index("s") * nc + lax.axis_index("c")
        base = wid * ew
        pltpu.sync_copy(sidx_hbm.at[pl.ds(base, ew)], sidx_v)
        pltpu.sync_copy(didx_hbm.at[pl.ds(base, ew)], didx_v)
        lanes = lax.iota(jnp.int32, _L)
        mneg = jnp.where(lanes == 0, -1.0, 1.0).astype(jnp.float32)

        def descs(c, sl):
            cs = pltpu.make_async_copy(
                z_hbm.at[sidx_v.at[pl.ds(c * chunk, chunk)]],
                srows_v.at[sl], sems_s[sl])
            cd = pltpu.make_async_copy(
                z_hbm.at[didx_v.at[pl.ds(c * chunk, chunk)]],
                drows_v.at[sl], sems_d[sl])
            return cs, cd

        def start(c, sl):
            cs, cd = descs(c, sl)
            cs.start()
            cd.start()

        def compute(c, sl):
            cs, cd = descs(c, sl)
            cs.wait()
            cd.wait()
            sref = srows_v.at[sl]
            dref = drows_v.at[sl]

            def do_group(g, carry2):
                ebase = g * _L
                for l in range(_L):
                    e = ebase + l
                    ps = []
                    for k in range(dw // _L):
                        sv = sref[e, pl.ds(k * _L, _L)]
                        dv = dref[e, pl.ds(k * _L, _L)]
                        pb = (plsc.bitcast(sv, jnp.bfloat16)
                              * plsc.bitcast(dv, jnp.bfloat16))
                        pa, pc = plsc.unpack(
                            pb, format=plsc.PackFormat.INTERLEAVED)
                        if k == 0:
                            pa = pa * mneg
                        ps += [pa, pc]
                    while len(ps) > 1:
                        ps = [ps[i] + ps[i + 1] for i in range(0, len(ps), 2)]
                    tbuf_v[l, pl.ds(0, _L)] = ps[0]
                one = jnp.full((_L,), 1, jnp.int32)
                col = jnp.zeros((_L,), jnp.int32)
                zero = jnp.zeros((_L,), jnp.float32)
                accs = [zero, zero, zero, zero]
                for k in range(_L):
                    accs[k % 4] = accs[k % 4] + plsc.load_gather(
                        tbuf_v, [lanes, col])
                    col = col + one
                res = (accs[0] + accs[1]) + (accs[2] + accs[3])
                out_v[pl.ds(c * chunk + ebase, _L)] = res
                return carry2

            lax.fori_loop(0, chunk // _L, do_group, 0)

        for c in range(nslot - 1):
            start(c, c)

        def step(i, carry):
            for b in range(nslot):
                c = nslot * i + b

                next_slot = (b + nslot - 1) % nslot

                @pl.when(c + nslot - 1 < nchunk)
                def _():
                    start(c + nslot - 1, next_slot)

                compute(c, b)
            return carry

        nfull = nchunk // nslot
        lax.fori_loop(0, nfull, step, 0)
        for c in range(nfull * nslot, nchunk):
            compute(c, c % nslot)

        pltpu.sync_copy(out_v, out_hbm.at[pl.ds(base, ew)])

    return body(zp, src_idx, dst_idx)


def _tc_decode_body(r_ref, t_ref, inner_ref, o_ref):
    inner = inner_ref[...]
    arg = jnp.maximum(-inner, 1.0 + 1e-7)
    dist = jnp.log(arg + jnp.sqrt(arg * arg - 1.0))
    o_ref[...] = 1.0 / (jnp.exp((dist - r_ref[0, 0]) / t_ref[0, 0]) + 1.0)


def _tc_decode(inner2d, r, t):
    rows, cols = inner2d.shape
    return pl.pallas_call(
        _tc_decode_body,
        out_shape=jax.ShapeDtypeStruct((rows, cols), jnp.float32),
        in_specs=[
            pl.BlockSpec(memory_space=pltpu.SMEM),
            pl.BlockSpec(memory_space=pltpu.SMEM),
            pl.BlockSpec(memory_space=pltpu.VMEM),
        ],
        out_specs=pl.BlockSpec(memory_space=pltpu.VMEM),
    )(r.reshape(1, 1).astype(jnp.float32), t.reshape(1, 1).astype(jnp.float32),
      inner2d)


def kernel(z, edge_index, r, t):
    ei = edge_index.astype(jnp.int32)
    n, d = z.shape
    zp = jax.lax.bitcast_convert_type(
        z.astype(jnp.bfloat16).reshape(n, d // 2, 2), jnp.int32)
    inner = _sc_inner_products(zp, ei[0], ei[1])
    e_total = inner.shape[0]
    probs2d = _tc_decode(inner.reshape(e_total // 128, 128), r, t)
    return probs2d.reshape(e_total)
